# baseline (speedup 1.0000x reference)
import functools

import jax
import jax.numpy as jnp
from jax.experimental import pallas as pl

_ROW_BLOCK = 128


def _body(h_ref, w1_ref, b1_ref, w2_ref, b2_ref, gamma_ref, beta_ref,
          out_ref, sparse_ref, *, k):
    h = h_ref[...]
    z = jnp.dot(h, w1_ref[...], preferred_element_type=jnp.float32)
    z = z + b1_ref[...]
    e = 0.5 * z * (1.0 + jax.lax.erf(z * 0.7071067811865476))

    b = jax.lax.bitcast_convert_type(e, jnp.int32)
    m = jnp.where(b >= 0, b, b ^ jnp.int32(0x7FFFFFFF))

    one16 = jnp.int16(1)
    zero16 = jnp.int16(0)
    k32 = jnp.float32(k)

    def count_ge(x, cand):
        v = jnp.where(x >= cand, one16, zero16)
        w = v.shape[1]
        while w > 128:
            w //= 2
            v = v[:, :w] + v[:, w:]
        return jnp.sum(v.astype(jnp.float32), axis=1, keepdims=True)

    m_hi = (m >> 16).astype(jnp.int16)
    cnt0 = count_ge(m_hi, zero16)
    base_hi = jnp.where(cnt0 >= k32, jnp.int32(0), jnp.int32(-32768))
    for bit in range(14, -1, -1):
        cand = base_hi | jnp.int32(1 << bit)
        cnt = count_ge(m_hi, cand.astype(jnp.int16))
        base_hi = jnp.where(cnt >= k32, cand, base_hi)
    base_hi16 = base_hi.astype(jnp.int16)

    cnt_above = count_ge(m_hi, (base_hi + 1).astype(jnp.int16))
    band = m_hi == base_hi16
    m_lo = (m ^ jnp.int32(0x8000)).astype(jnp.int16)
    ml = jnp.where(band, m_lo, jnp.int16(-32768))
    kk = k32 - cnt_above
    base_lo = jnp.where(count_ge(ml, zero16) >= kk,
                        jnp.int32(0), jnp.int32(-32768))
    for bit in range(14, -1, -1):
        cand = base_lo | jnp.int32(1 << bit)
        cnt = count_ge(ml, cand.astype(jnp.int16))
        base_lo = jnp.where(cnt >= kk, cand, base_lo)

    thresh = (base_hi << 16) | ((base_lo ^ 0x8000) & 0xFFFF)
    sp = jnp.where(m >= thresh, e, 0.0)
    sparse_ref[...] = sp

    c = jnp.dot(sp.astype(jnp.bfloat16), w2_ref[...],
                preferred_element_type=jnp.float32)
    x = c + b2_ref[...] + h
    mu = jnp.mean(x, axis=1, keepdims=True)
    xc = x - mu
    var = jnp.mean(xc * xc, axis=1, keepdims=True)
    out_ref[...] = xc * jax.lax.rsqrt(var + 1e-5) * gamma_ref[...] + beta_ref[...]


def kernel(h_C, W1, b1, W2, b2, gamma, beta):
    B, T, D = h_C.shape
    E = W1.shape[1]
    k = max(1, int(E * 0.1))
    rows = B * T
    R = _ROW_BLOCK
    grid = rows // R

    h2 = h_C.reshape(rows, D)
    w2b = W2.astype(jnp.bfloat16)

    out, sparse = pl.pallas_call(
        functools.partial(_body, k=k),
        grid=(grid,),
        in_specs=[
            pl.BlockSpec((R, D), lambda i: (i, 0)),
            pl.BlockSpec((D, E), lambda i: (0, 0)),
            pl.BlockSpec((1, E), lambda i: (0, 0)),
            pl.BlockSpec((E, D), lambda i: (0, 0)),
            pl.BlockSpec((1, D), lambda i: (0, 0)),
            pl.BlockSpec((1, D), lambda i: (0, 0)),
            pl.BlockSpec((1, D), lambda i: (0, 0)),
        ],
        out_specs=[
            pl.BlockSpec((R, D), lambda i: (i, 0)),
            pl.BlockSpec((R, E), lambda i: (i, 0)),
        ],
        out_shape=[
            jax.ShapeDtypeStruct((rows, D), jnp.float32),
            jax.ShapeDtypeStruct((rows, E), jnp.float32),
        ],
    )(h2, W1, b1.reshape(1, E), w2b, b2.reshape(1, D),
      gamma.reshape(1, D), beta.reshape(1, D))

    return out.reshape(B, T, D), sparse.reshape(B, T, E)


# speedup vs baseline: 1.0351x; 1.0351x over previous
import functools

import jax
import jax.numpy as jnp
from jax.experimental import pallas as pl

_ROW_BLOCK = 256


def _body(h_ref, w1_ref, b1_ref, w2_ref, b2_ref, gamma_ref, beta_ref,
          out_ref, sparse_ref, *, k):
    h = h_ref[...]
    z = jnp.dot(h, w1_ref[...], preferred_element_type=jnp.float32)
    z = z + b1_ref[...]
    e = 0.5 * z * (1.0 + jax.lax.erf(z * 0.7071067811865476))

    b = jax.lax.bitcast_convert_type(e, jnp.int32)
    m = jnp.where(b >= 0, b, b ^ jnp.int32(0x7FFFFFFF))

    one16 = jnp.int16(1)
    zero16 = jnp.int16(0)
    k32 = jnp.float32(k)

    def count_ge(x, cand):
        w = x.shape[1]
        if w >= 1024:
            parts = [jnp.where(x[:, j * 512:(j + 1) * 512] >= cand,
                               one16, zero16) for j in range(w // 512)]
            while len(parts) > 1:
                parts = [parts[i] + parts[i + 1]
                         for i in range(0, len(parts), 2)]
            v = parts[0]
            w = 512
        else:
            v = jnp.where(x >= cand, one16, zero16)
        while w > 128:
            w //= 2
            v = v[:, :w] + v[:, w:]
        return jnp.sum(v.astype(jnp.float32), axis=1, keepdims=True)

    m_hi = (m >> 16).astype(jnp.int16)
    cnt0 = count_ge(m_hi, zero16)
    base_hi = jnp.where(cnt0 >= k32, jnp.int32(0), jnp.int32(-32768))
    for bit in range(14, -1, -1):
        cand = base_hi | jnp.int32(1 << bit)
        cnt = count_ge(m_hi, cand.astype(jnp.int16))
        base_hi = jnp.where(cnt >= k32, cand, base_hi)
    base_hi16 = base_hi.astype(jnp.int16)

    cnt_above = count_ge(m_hi, (base_hi + 1).astype(jnp.int16))
    band = m_hi == base_hi16
    m_lo = (m ^ jnp.int32(0x8000)).astype(jnp.int16)
    ml = jnp.where(band, m_lo, jnp.int16(-32768))
    kk = k32 - cnt_above
    base_lo = jnp.where(count_ge(ml, zero16) >= kk,
                        jnp.int32(0), jnp.int32(-32768))
    for bit in range(14, -1, -1):
        cand = base_lo | jnp.int32(1 << bit)
        cnt = count_ge(ml, cand.astype(jnp.int16))
        base_lo = jnp.where(cnt >= kk, cand, base_lo)

    thresh = (base_hi << 16) | ((base_lo ^ 0x8000) & 0xFFFF)
    sp = jnp.where(m >= thresh, e, 0.0)
    sparse_ref[...] = sp

    c = jnp.dot(sp.astype(jnp.bfloat16), w2_ref[...],
                preferred_element_type=jnp.float32)
    x = c + b2_ref[...] + h
    mu = jnp.mean(x, axis=1, keepdims=True)
    xc = x - mu
    var = jnp.mean(xc * xc, axis=1, keepdims=True)
    out_ref[...] = xc * jax.lax.rsqrt(var + 1e-5) * gamma_ref[...] + beta_ref[...]


def kernel(h_C, W1, b1, W2, b2, gamma, beta):
    B, T, D = h_C.shape
    E = W1.shape[1]
    k = max(1, int(E * 0.1))
    rows = B * T
    R = _ROW_BLOCK
    grid = rows // R

    h2 = h_C.reshape(rows, D)
    w2b = W2.astype(jnp.bfloat16)

    out, sparse = pl.pallas_call(
        functools.partial(_body, k=k),
        grid=(grid,),
        in_specs=[
            pl.BlockSpec((R, D), lambda i: (i, 0)),
            pl.BlockSpec((D, E), lambda i: (0, 0)),
            pl.BlockSpec((1, E), lambda i: (0, 0)),
            pl.BlockSpec((E, D), lambda i: (0, 0)),
            pl.BlockSpec((1, D), lambda i: (0, 0)),
            pl.BlockSpec((1, D), lambda i: (0, 0)),
            pl.BlockSpec((1, D), lambda i: (0, 0)),
        ],
        out_specs=[
            pl.BlockSpec((R, D), lambda i: (i, 0)),
            pl.BlockSpec((R, E), lambda i: (i, 0)),
        ],
        out_shape=[
            jax.ShapeDtypeStruct((rows, D), jnp.float32),
            jax.ShapeDtypeStruct((rows, E), jnp.float32),
        ],
    )(h2, W1, b1.reshape(1, E), w2b, b2.reshape(1, D),
      gamma.reshape(1, D), beta.reshape(1, D))

    return out.reshape(B, T, D), sparse.reshape(B, T, E)
